# baseline (device time: 84471 ns/iter reference)
import jax
import jax.numpy as jnp
from jax import lax
from jax.experimental import pallas as pl
from jax.experimental.pallas import tpu as pltpu

N_DEV = 4
SQ = 256
SKV = 4096
H_PER = 8
DH = 128
D_MODEL = 1024
HALF = D_MODEL // 2
SCALE = 0.08838834764831843

BLK = 64
N_KB = SKV // BLK

def _kept(qb):
    return sorted({0, qb} | {kb for kb in range(N_KB) if (qb + kb) % 3 == 0})

_GROUPS = [
    ([(0, 64), (192, 64)], _kept(0)),
    ([(64, 64)], _kept(1)),
    ([(128, 64)], _kept(2)),
]
_G_NKV = [len(b) * BLK for _, b in _GROUPS]
_G_OFF = [sum(_G_NKV[:g]) for g in range(len(_GROUPS))]
_G_TOT = sum(_G_NKV)
assert _kept(0) == _kept(3)


def kernel(x, Wq, K_ext, V_ext, Wo):
    k2 = K_ext.reshape(SKV, H_PER * DH)
    v2 = V_ext.reshape(SKV, H_PER * DH)

    def body(x_ref, wq_hbm, k_hbm, v_hbm, wo_hbm, out_ref,
             wq_buf, k_buf, v_buf, wo_buf, comm_l, comm_r,
             load_sems, send_l, recv_l, send_r, recv_r):
        my_pos = lax.axis_index("i")
        left = (my_pos - 1) % N_DEV
        right = (my_pos + 1) % N_DEV

        col0 = my_pos * (H_PER * DH)

        cp_wq = pltpu.make_async_copy(
            wq_hbm.at[:, pl.ds(col0, H_PER * DH)], wq_buf, load_sems.at[0])
        cp_wo = pltpu.make_async_copy(
            wo_hbm.at[pl.ds(col0, H_PER * DH), :], wo_buf, load_sems.at[3])
        kv_cps = []
        for g, (_, blocks) in enumerate(_GROUPS):
            for j, kbk in enumerate(blocks):
                dst = _G_OFF[g] + j * BLK
                kv_cps.append(pltpu.make_async_copy(
                    k_hbm.at[pl.ds(kbk * BLK, BLK), :],
                    k_buf.at[pl.ds(dst, BLK), :],
                    load_sems.at[1]))
                kv_cps.append(pltpu.make_async_copy(
                    v_hbm.at[pl.ds(kbk * BLK, BLK), :],
                    v_buf.at[pl.ds(dst, BLK), :],
                    load_sems.at[2]))
        cp_wq.start()
        for cp in kv_cps:
            cp.start()
        cp_wo.start()

        barrier_sem = pltpu.get_barrier_semaphore()
        for nbr in [left, right]:
            pl.semaphore_signal(
                barrier_sem, inc=1,
                device_id=(nbr,), device_id_type=pl.DeviceIdType.MESH,
            )
        pl.semaphore_wait(barrier_sem, 2)

        xm = x_ref[0]

        cp_wq.wait()
        q_all = jnp.dot(xm, wq_buf[...],
                        preferred_element_type=jnp.float32)
        for cp in kv_cps:
            cp.wait()

        ctx_heads = []
        for h in range(H_PER):
            hc = slice(h * DH, (h + 1) * DH)
            qh = q_all[:, hc]
            ctx_parts = {}
            for g, (rows, _) in enumerate(_GROUPS):
                qg = jnp.concatenate(
                    [qh[r0:r0 + n] for r0, n in rows], axis=0)
                kg = k_buf[_G_OFF[g]:_G_OFF[g] + _G_NKV[g], hc]
                vg = v_buf[_G_OFF[g]:_G_OFF[g] + _G_NKV[g], hc]
                s = lax.dot_general(
                    qg, kg, (((1,), (1,)), ((), ())),
                    preferred_element_type=jnp.float32,
                ) * SCALE
                m = jnp.max(s, axis=1, keepdims=True)
                p = jnp.exp(s - m)
                w = p / jnp.sum(p, axis=1, keepdims=True)
                ctx_parts[g] = jnp.dot(
                    w, vg, preferred_element_type=jnp.float32)
            ctx_heads.append(jnp.concatenate([
                ctx_parts[0][0:64],
                ctx_parts[1],
                ctx_parts[2],
                ctx_parts[0][64:128],
            ], axis=0))
        ctx_all = jnp.concatenate(ctx_heads, axis=1)

        cp_wo.wait()
        acc = jnp.dot(ctx_all, wo_buf[...],
                      preferred_element_type=jnp.float32)

        acc_l = acc[:, :HALF]
        acc_r = acc[:, HALF:]
        comm_l[0] = acc_l
        comm_r[0] = acc_r
        for h in range(N_DEV - 1):
            rdma_l = pltpu.make_async_remote_copy(
                src_ref=comm_l.at[h], dst_ref=comm_l.at[h + 1],
                send_sem=send_l.at[h], recv_sem=recv_l.at[h + 1],
                device_id=(right,), device_id_type=pl.DeviceIdType.MESH,
            )
            rdma_r = pltpu.make_async_remote_copy(
                src_ref=comm_r.at[h], dst_ref=comm_r.at[h + 1],
                send_sem=send_r.at[h], recv_sem=recv_r.at[h + 1],
                device_id=(left,), device_id_type=pl.DeviceIdType.MESH,
            )
            rdma_l.start()
            rdma_r.start()
            if h > 0:
                acc_l = acc_l + comm_l[h]
                acc_r = acc_r + comm_r[h]
            rdma_l.wait()
            rdma_r.wait()
        acc_l = acc_l + comm_l[N_DEV - 1]
        acc_r = acc_r + comm_r[N_DEV - 1]

        out_ref[0] = jnp.concatenate([acc_l, acc_r], axis=1)

    out_shape = jax.ShapeDtypeStruct((1, SQ, D_MODEL), jnp.float32)
    return pl.pallas_call(
        body,
        out_shape=out_shape,
        in_specs=[
            pl.BlockSpec(memory_space=pltpu.VMEM),
            pl.BlockSpec(memory_space=pl.ANY),
            pl.BlockSpec(memory_space=pl.ANY),
            pl.BlockSpec(memory_space=pl.ANY),
            pl.BlockSpec(memory_space=pl.ANY),
        ],
        out_specs=pl.BlockSpec(memory_space=pltpu.VMEM),
        scratch_shapes=[
            pltpu.VMEM((D_MODEL, H_PER * DH), jnp.float32),
            pltpu.VMEM((_G_TOT, H_PER * DH), jnp.float32),
            pltpu.VMEM((_G_TOT, H_PER * DH), jnp.float32),
            pltpu.VMEM((H_PER * DH, D_MODEL), jnp.float32),
            pltpu.VMEM((N_DEV, SQ, HALF), jnp.float32),
            pltpu.VMEM((N_DEV, SQ, HALF), jnp.float32),
            pltpu.SemaphoreType.DMA((4,)),
            pltpu.SemaphoreType.DMA((N_DEV,)),
            pltpu.SemaphoreType.DMA((N_DEV,)),
            pltpu.SemaphoreType.DMA((N_DEV,)),
            pltpu.SemaphoreType.DMA((N_DEV,)),
        ],
        compiler_params=pltpu.CompilerParams(
            collective_id=0,
            vmem_limit_bytes=61 * 1024 * 1024,
        ),
    )(x, Wq, k2, v2, Wo)


# device time: 57584 ns/iter; 1.4669x vs baseline; 1.4669x over previous
import jax
import jax.numpy as jnp
from jax import lax
from jax.experimental import pallas as pl
from jax.experimental.pallas import tpu as pltpu

N_DEV = 4
SQ = 256
SKV = 4096
H_PER = 8
DH = 128
D_MODEL = 1024
HALF = D_MODEL // 2
SCALE = 0.08838834764831843

BLK = 64
N_KB = SKV // BLK

def _kept(qb):
    return sorted({0, qb} | {kb for kb in range(N_KB) if (qb + kb) % 3 == 0})

_GROUPS = [
    ([(0, 64), (192, 64)], _kept(0)),
    ([(64, 64)], _kept(1)),
    ([(128, 64)], _kept(2)),
]
_G_NKV = [len(b) * BLK for _, b in _GROUPS]
_G_OFF = [sum(_G_NKV[:g]) for g in range(len(_GROUPS))]
_G_TOT = sum(_G_NKV)
assert _kept(0) == _kept(3)


def kernel(x, Wq, K_ext, V_ext, Wo):
    def body(x_ref, wq_hbm, k4_hbm, v4_hbm, wo_hbm, out_ref,
             wq_buf, k_buf, v_buf, wo_buf, comm_l, comm_r,
             load_sems, send_l, recv_l, send_r, recv_r):
        my_pos = lax.axis_index("i")
        left = (my_pos - 1) % N_DEV
        right = (my_pos + 1) % N_DEV

        col0 = my_pos * (H_PER * DH)

        k_hbm = k4_hbm.at[0].reshape(SKV, H_PER * DH)
        v_hbm = v4_hbm.at[0].reshape(SKV, H_PER * DH)

        cp_wq = pltpu.make_async_copy(
            wq_hbm.at[:, pl.ds(col0, H_PER * DH)], wq_buf, load_sems.at[0])
        cp_wo = pltpu.make_async_copy(
            wo_hbm.at[pl.ds(col0, H_PER * DH), :], wo_buf, load_sems.at[3])
        kv_cps = []
        for g, (_, blocks) in enumerate(_GROUPS):
            for j, kbk in enumerate(blocks):
                dst = _G_OFF[g] + j * BLK
                kv_cps.append(pltpu.make_async_copy(
                    k_hbm.at[pl.ds(kbk * BLK, BLK), :],
                    k_buf.at[pl.ds(dst, BLK), :],
                    load_sems.at[1]))
                kv_cps.append(pltpu.make_async_copy(
                    v_hbm.at[pl.ds(kbk * BLK, BLK), :],
                    v_buf.at[pl.ds(dst, BLK), :],
                    load_sems.at[2]))
        cp_wq.start()
        for cp in kv_cps:
            cp.start()
        cp_wo.start()

        barrier_sem = pltpu.get_barrier_semaphore()
        for nbr in [left, right]:
            pl.semaphore_signal(
                barrier_sem, inc=1,
                device_id=(nbr,), device_id_type=pl.DeviceIdType.MESH,
            )
        pl.semaphore_wait(barrier_sem, 2)

        xm = x_ref[0]

        cp_wq.wait()
        q_all = jnp.dot(xm, wq_buf[...],
                        preferred_element_type=jnp.float32)
        for cp in kv_cps:
            cp.wait()

        ctx_heads = []
        for h in range(H_PER):
            hc = slice(h * DH, (h + 1) * DH)
            qh = q_all[:, hc]
            ctx_parts = {}
            for g, (rows, _) in enumerate(_GROUPS):
                qg = jnp.concatenate(
                    [qh[r0:r0 + n] for r0, n in rows], axis=0)
                kg = k_buf[_G_OFF[g]:_G_OFF[g] + _G_NKV[g], hc]
                vg = v_buf[_G_OFF[g]:_G_OFF[g] + _G_NKV[g], hc]
                s = lax.dot_general(
                    qg, kg, (((1,), (1,)), ((), ())),
                    preferred_element_type=jnp.float32,
                ) * SCALE
                m = jnp.max(s, axis=1, keepdims=True)
                p = jnp.exp(s - m)
                w = p / jnp.sum(p, axis=1, keepdims=True)
                ctx_parts[g] = jnp.dot(
                    w, vg, preferred_element_type=jnp.float32)
            ctx_heads.append(jnp.concatenate([
                ctx_parts[0][0:64],
                ctx_parts[1],
                ctx_parts[2],
                ctx_parts[0][64:128],
            ], axis=0))
        ctx_all = jnp.concatenate(ctx_heads, axis=1)

        cp_wo.wait()
        acc = jnp.dot(ctx_all, wo_buf[...],
                      preferred_element_type=jnp.float32)

        acc_l = acc[:, :HALF]
        acc_r = acc[:, HALF:]
        comm_l[0] = acc_l
        comm_r[0] = acc_r
        for h in range(N_DEV - 1):
            rdma_l = pltpu.make_async_remote_copy(
                src_ref=comm_l.at[h], dst_ref=comm_l.at[h + 1],
                send_sem=send_l.at[h], recv_sem=recv_l.at[h + 1],
                device_id=(right,), device_id_type=pl.DeviceIdType.MESH,
            )
            rdma_r = pltpu.make_async_remote_copy(
                src_ref=comm_r.at[h], dst_ref=comm_r.at[h + 1],
                send_sem=send_r.at[h], recv_sem=recv_r.at[h + 1],
                device_id=(left,), device_id_type=pl.DeviceIdType.MESH,
            )
            rdma_l.start()
            rdma_r.start()
            if h > 0:
                acc_l = acc_l + comm_l[h]
                acc_r = acc_r + comm_r[h]
            rdma_l.wait()
            rdma_r.wait()
        acc_l = acc_l + comm_l[N_DEV - 1]
        acc_r = acc_r + comm_r[N_DEV - 1]

        out_ref[0] = jnp.concatenate([acc_l, acc_r], axis=1)

    out_shape = jax.ShapeDtypeStruct((1, SQ, D_MODEL), jnp.float32)
    return pl.pallas_call(
        body,
        out_shape=out_shape,
        in_specs=[
            pl.BlockSpec(memory_space=pltpu.VMEM),
            pl.BlockSpec(memory_space=pl.ANY),
            pl.BlockSpec(memory_space=pl.ANY),
            pl.BlockSpec(memory_space=pl.ANY),
            pl.BlockSpec(memory_space=pl.ANY),
        ],
        out_specs=pl.BlockSpec(memory_space=pltpu.VMEM),
        scratch_shapes=[
            pltpu.VMEM((D_MODEL, H_PER * DH), jnp.float32),
            pltpu.VMEM((_G_TOT, H_PER * DH), jnp.float32),
            pltpu.VMEM((_G_TOT, H_PER * DH), jnp.float32),
            pltpu.VMEM((H_PER * DH, D_MODEL), jnp.float32),
            pltpu.VMEM((N_DEV, SQ, HALF), jnp.float32),
            pltpu.VMEM((N_DEV, SQ, HALF), jnp.float32),
            pltpu.SemaphoreType.DMA((4,)),
            pltpu.SemaphoreType.DMA((N_DEV,)),
            pltpu.SemaphoreType.DMA((N_DEV,)),
            pltpu.SemaphoreType.DMA((N_DEV,)),
            pltpu.SemaphoreType.DMA((N_DEV,)),
        ],
        compiler_params=pltpu.CompilerParams(
            collective_id=0,
            vmem_limit_bytes=61 * 1024 * 1024,
        ),
    )(x, Wq, K_ext, V_ext, Wo)


# device time: 37271 ns/iter; 2.2664x vs baseline; 1.5450x over previous
import jax
import jax.numpy as jnp
from jax import lax
from jax.experimental import pallas as pl
from jax.experimental.pallas import tpu as pltpu

N_DEV = 4
SQ = 256
SKV = 4096
H_PER = 8
DH = 128
D_MODEL = 1024
CH = D_MODEL // N_DEV
SCALE = 0.08838834764831843

BLK = 64
N_KB = SKV // BLK

def _kept(qb):
    return sorted({0, qb} | {kb for kb in range(N_KB) if (qb + kb) % 3 == 0})

_GROUPS = [
    ([(0, 64), (192, 64)], _kept(0)),
    ([(64, 64)], _kept(1)),
    ([(128, 64)], _kept(2)),
]
_G_NKV = [len(b) * BLK for _, b in _GROUPS]
_G_OFF = [sum(_G_NKV[:g]) for g in range(len(_GROUPS))]
_G_TOT = sum(_G_NKV)
assert _kept(0) == _kept(3)


def kernel(x, Wq, K_ext, V_ext, Wo):
    def body(x_ref, wq_hbm, k4_hbm, v4_hbm, wo_hbm, out_ref,
             wq_buf, k_buf, v_buf, wo_buf,
             rs_src, rs_buf, ag_src, ag_buf,
             load_sems, rs_send, rs_recv, ag_send, ag_recv):
        my_pos = lax.axis_index("i")

        col0 = my_pos * (H_PER * DH)

        k_hbm = k4_hbm.at[0].reshape(SKV, H_PER * DH)
        v_hbm = v4_hbm.at[0].reshape(SKV, H_PER * DH)

        cp_wq = pltpu.make_async_copy(
            wq_hbm.at[:, pl.ds(col0, H_PER * DH)], wq_buf, load_sems.at[0])
        cp_wo = pltpu.make_async_copy(
            wo_hbm.at[pl.ds(col0, H_PER * DH), :], wo_buf, load_sems.at[1])
        kv_cps = [[] for _ in _GROUPS]
        for g, (_, blocks) in enumerate(_GROUPS):
            for j, kbk in enumerate(blocks):
                dst = _G_OFF[g] + j * BLK
                kv_cps[g].append(pltpu.make_async_copy(
                    k_hbm.at[pl.ds(kbk * BLK, BLK), :],
                    k_buf.at[pl.ds(dst, BLK), :],
                    load_sems.at[2 + g]))
                kv_cps[g].append(pltpu.make_async_copy(
                    v_hbm.at[pl.ds(kbk * BLK, BLK), :],
                    v_buf.at[pl.ds(dst, BLK), :],
                    load_sems.at[5 + g]))
        cp_wq.start()
        for cps in kv_cps:
            for cp in cps:
                cp.start()
        cp_wo.start()

        barrier_sem = pltpu.get_barrier_semaphore()
        for d in range(1, N_DEV):
            pl.semaphore_signal(
                barrier_sem, inc=1,
                device_id=((my_pos + d) % N_DEV,),
                device_id_type=pl.DeviceIdType.MESH,
            )
        pl.semaphore_wait(barrier_sem, N_DEV - 1)

        xm = x_ref[0]

        cp_wq.wait()
        q_all = jnp.dot(xm, wq_buf[...],
                        preferred_element_type=jnp.float32)

        ctx_parts = {}
        for g, (rows, _) in enumerate(_GROUPS):
            for cp in kv_cps[g]:
                cp.wait()
            kg = k_buf[_G_OFF[g]:_G_OFF[g] + _G_NKV[g], :]
            vg = v_buf[_G_OFF[g]:_G_OFF[g] + _G_NKV[g], :]
            for h in range(H_PER):
                hc = slice(h * DH, (h + 1) * DH)
                qg = jnp.concatenate(
                    [q_all[r0:r0 + n, hc] for r0, n in rows], axis=0)
                s = lax.dot_general(
                    qg, kg[:, hc], (((1,), (1,)), ((), ())),
                    preferred_element_type=jnp.float32,
                ) * SCALE
                m = jnp.max(s, axis=1, keepdims=True)
                p = jnp.exp(s - m)
                w = p / jnp.sum(p, axis=1, keepdims=True)
                ctx_parts[g, h] = jnp.dot(
                    w, vg[:, hc], preferred_element_type=jnp.float32)
        ctx_all = jnp.concatenate([
            jnp.concatenate([
                ctx_parts[0, h][0:64],
                ctx_parts[1, h],
                ctx_parts[2, h],
                ctx_parts[0, h][64:128],
            ], axis=0)
            for h in range(H_PER)
        ], axis=1)

        cp_wo.wait()
        acc = jnp.dot(ctx_all, wo_buf[...],
                      preferred_element_type=jnp.float32)

        rs_src[...] = acc.astype(jnp.bfloat16)
        rs_rdmas = []
        for d in range(1, N_DEV):
            tgt = (my_pos + d) % N_DEV
            rdma = pltpu.make_async_remote_copy(
                src_ref=rs_src.at[:, pl.ds(tgt * CH, CH)],
                dst_ref=rs_buf.at[d - 1],
                send_sem=rs_send.at[d - 1],
                recv_sem=rs_recv.at[d - 1],
                device_id=(tgt,),
                device_id_type=pl.DeviceIdType.MESH,
            )
            rdma.start()
            rs_rdmas.append(rdma)
        for rdma in rs_rdmas:
            rdma.wait()

        red = rs_src[:, pl.ds(my_pos * CH, CH)].astype(jnp.float32)
        for sslot in range(N_DEV - 1):
            red = red + rs_buf[sslot].astype(jnp.float32)

        ag_src[...] = red.astype(jnp.bfloat16)
        ag_rdmas = []
        for d in range(1, N_DEV):
            tgt = (my_pos + d) % N_DEV
            rdma = pltpu.make_async_remote_copy(
                src_ref=ag_src,
                dst_ref=ag_buf.at[d - 1],
                send_sem=ag_send.at[d - 1],
                recv_sem=ag_recv.at[d - 1],
                device_id=(tgt,),
                device_id_type=pl.DeviceIdType.MESH,
            )
            rdma.start()
            ag_rdmas.append(rdma)
        out_ref[0, :, pl.ds(my_pos * CH, CH)] = red
        for d, rdma in zip(range(1, N_DEV), ag_rdmas):
            rdma.wait()
            p = (my_pos - d) % N_DEV
            out_ref[0, :, pl.ds(p * CH, CH)] = (
                ag_buf[d - 1].astype(jnp.float32))

    out_shape = jax.ShapeDtypeStruct((1, SQ, D_MODEL), jnp.float32)
    return pl.pallas_call(
        body,
        out_shape=out_shape,
        in_specs=[
            pl.BlockSpec(memory_space=pltpu.VMEM),
            pl.BlockSpec(memory_space=pl.ANY),
            pl.BlockSpec(memory_space=pl.ANY),
            pl.BlockSpec(memory_space=pl.ANY),
            pl.BlockSpec(memory_space=pl.ANY),
        ],
        out_specs=pl.BlockSpec(memory_space=pltpu.VMEM),
        scratch_shapes=[
            pltpu.VMEM((D_MODEL, H_PER * DH), jnp.float32),
            pltpu.VMEM((_G_TOT, H_PER * DH), jnp.float32),
            pltpu.VMEM((_G_TOT, H_PER * DH), jnp.float32),
            pltpu.VMEM((H_PER * DH, D_MODEL), jnp.float32),
            pltpu.VMEM((SQ, D_MODEL), jnp.bfloat16),
            pltpu.VMEM((N_DEV - 1, SQ, CH), jnp.bfloat16),
            pltpu.VMEM((SQ, CH), jnp.bfloat16),
            pltpu.VMEM((N_DEV - 1, SQ, CH), jnp.bfloat16),
            pltpu.SemaphoreType.DMA((8,)),
            pltpu.SemaphoreType.DMA((N_DEV - 1,)),
            pltpu.SemaphoreType.DMA((N_DEV - 1,)),
            pltpu.SemaphoreType.DMA((N_DEV - 1,)),
            pltpu.SemaphoreType.DMA((N_DEV - 1,)),
        ],
        compiler_params=pltpu.CompilerParams(
            collective_id=0,
            vmem_limit_bytes=61 * 1024 * 1024,
        ),
    )(x, Wq, K_ext, V_ext, Wo)
